# baseline (speedup 1.0000x reference)
import jax
import jax.numpy as jnp
from jax.experimental import pallas as pl
from jax.experimental.pallas import tpu as pltpu


_N_CHUNKS = 8
_HEAD_ROWS = 8


def _body(x_ref, y_ref, o_ref, head_x, head_y, copy_sems, head_sem):
    n_rows = x_ref.shape[0]
    chunk = (n_rows - _HEAD_ROWS) // _N_CHUNKS

    copies = []
    for k in range(_N_CHUNKS):
        ds = pl.ds(_HEAD_ROWS + k * chunk, chunk)
        cp = pltpu.make_async_copy(x_ref.at[ds, :], o_ref.at[ds, :],
                                   copy_sems.at[k])
        cp.start()
        copies.append(cp)

    hx = pltpu.make_async_copy(x_ref.at[pl.ds(0, _HEAD_ROWS), :], head_x,
                               head_sem)
    hx.start()
    hy = pltpu.make_async_copy(y_ref.at[pl.ds(0, _HEAD_ROWS), :], head_y,
                               head_sem)
    hy.start()
    hx.wait()
    hy.wait()
    col = jax.lax.broadcasted_iota(jnp.int32, (1, head_x.shape[1]), 1)
    head_x[0:1, :] = jnp.where(col == 3, head_y[0, 2], head_x[0:1, :])
    ho = pltpu.make_async_copy(head_x, o_ref.at[pl.ds(0, _HEAD_ROWS), :],
                               head_sem)
    ho.start()
    ho.wait()

    for cp in copies:
        cp.wait()


def kernel(x, y):
    n_rows, n_cols = x.shape
    return pl.pallas_call(
        _body,
        in_specs=[
            pl.BlockSpec(memory_space=pltpu.MemorySpace.HBM),
            pl.BlockSpec(memory_space=pltpu.MemorySpace.HBM),
        ],
        out_specs=pl.BlockSpec(memory_space=pltpu.MemorySpace.HBM),
        out_shape=jax.ShapeDtypeStruct(x.shape, x.dtype),
        scratch_shapes=[
            pltpu.VMEM((_HEAD_ROWS, n_cols), x.dtype),
            pltpu.VMEM((_HEAD_ROWS, n_cols), y.dtype),
            pltpu.SemaphoreType.DMA((_N_CHUNKS,)),
            pltpu.SemaphoreType.DMA,
        ],
    )(x, y)


# speedup vs baseline: 27.1837x; 27.1837x over previous
import jax
import jax.numpy as jnp
from jax.experimental import pallas as pl


_ROWS_PER_BLOCK = 2048


def _body(x_ref, y_ref, o_ref):
    o_ref[...] = x_ref[...]

    @pl.when(pl.program_id(0) == 0)
    def _patch():
        col = jax.lax.broadcasted_iota(jnp.int32, (1, 128), 1)
        o_ref[0:1, :] = jnp.where(col == 3, y_ref[0, 2], x_ref[0:1, :])


def kernel(x, y):
    n_rows, n_cols = x.shape
    grid = (n_rows // _ROWS_PER_BLOCK,)
    return pl.pallas_call(
        _body,
        grid=grid,
        in_specs=[
            pl.BlockSpec((_ROWS_PER_BLOCK, n_cols), lambda i: (i, 0)),
            pl.BlockSpec((8, n_cols), lambda i: (0, 0)),
        ],
        out_specs=pl.BlockSpec((_ROWS_PER_BLOCK, n_cols), lambda i: (i, 0)),
        out_shape=jax.ShapeDtypeStruct(x.shape, x.dtype),
    )(x, y)


# speedup vs baseline: 34.0613x; 1.2530x over previous
import jax
import jax.numpy as jnp
from jax.experimental import pallas as pl


_ROWS_PER_BLOCK = 4096


def _body(x_ref, y_ref, o_ref):
    o_ref[...] = x_ref[...]

    @pl.when(pl.program_id(0) == 0)
    def _patch():
        col = jax.lax.broadcasted_iota(jnp.int32, (1, 128), 1)
        o_ref[0:1, :] = jnp.where(col == 3, y_ref[0, 2], x_ref[0:1, :])


def kernel(x, y):
    n_rows, n_cols = x.shape
    grid = (n_rows // _ROWS_PER_BLOCK,)
    return pl.pallas_call(
        _body,
        grid=grid,
        in_specs=[
            pl.BlockSpec((_ROWS_PER_BLOCK, n_cols), lambda i: (i, 0)),
            pl.BlockSpec((8, n_cols), lambda i: (0, 0)),
        ],
        out_specs=pl.BlockSpec((_ROWS_PER_BLOCK, n_cols), lambda i: (i, 0)),
        out_shape=jax.ShapeDtypeStruct(x.shape, x.dtype),
    )(x, y)


# speedup vs baseline: 42.2859x; 1.2415x over previous
import jax
import jax.numpy as jnp
from jax.experimental import pallas as pl


_ROWS_PER_BLOCK = 8192


def _body(x_ref, y_ref, o_ref):
    o_ref[...] = x_ref[...]

    @pl.when(pl.program_id(0) == 0)
    def _patch():
        col = jax.lax.broadcasted_iota(jnp.int32, (1, 128), 1)
        o_ref[0:1, :] = jnp.where(col == 3, y_ref[0, 2], x_ref[0:1, :])


def kernel(x, y):
    n_rows, n_cols = x.shape
    grid = (n_rows // _ROWS_PER_BLOCK,)
    return pl.pallas_call(
        _body,
        grid=grid,
        in_specs=[
            pl.BlockSpec((_ROWS_PER_BLOCK, n_cols), lambda i: (i, 0)),
            pl.BlockSpec((8, n_cols), lambda i: (0, 0)),
        ],
        out_specs=pl.BlockSpec((_ROWS_PER_BLOCK, n_cols), lambda i: (i, 0)),
        out_shape=jax.ShapeDtypeStruct(x.shape, x.dtype),
    )(x, y)
